# baseline (speedup 1.0000x reference)
import jax
import jax.numpy as jnp
from jax.experimental import pallas as pl
from jax.experimental.pallas import tpu as pltpu

_TOPK = 8


def _moe_kernel(x_ref, gw_ref, ga_ref, ub_ref, dwa_ref, dwb_ref,
                sgu_ref, sdw_ref, out_ref, comb_ref, xb_ref):
    i = pl.program_id(0)
    T = x_ref.shape[0]

    @pl.when(i == 0)
    def _prologue():
        x = x_ref[...]
        xb_ref[...] = x.astype(jnp.bfloat16)

        gw = gw_ref[...]
        logits = jax.lax.dot_general(
            x, gw, (((1,), (1,)), ((), ())),
            preferred_element_type=jnp.float32)
        m = jnp.max(logits, axis=-1, keepdims=True)
        ex = jnp.exp(logits - m)
        probs = ex / jnp.sum(ex, axis=-1, keepdims=True)
        remaining = probs
        comb = jnp.zeros(probs.shape, jnp.float32)
        n_e = probs.shape[1]
        lane = jax.lax.broadcasted_iota(jnp.int32, probs.shape, 1)
        for _ in range(_TOPK):
            mx = jnp.max(remaining, axis=-1, keepdims=True)
            ismax = remaining == mx
            first_idx = jnp.min(
                jnp.where(ismax, lane, n_e), axis=-1, keepdims=True)
            first = lane == first_idx
            comb = comb + jnp.where(first, remaining, 0.0)
            remaining = jnp.where(first, -jnp.inf, remaining)
        comb_ref[...] = comb

        sgu = jax.lax.dot_general(
            x, sgu_ref[...], (((1,), (1,)), ((), ())),
            preferred_element_type=jnp.float32)
        inter = sdw_ref.shape[1]
        g = sgu[:, :inter]
        u = sgu[:, inter:]
        sh = g * jax.nn.sigmoid(g) * u
        out_ref[...] = jax.lax.dot_general(
            sh, sdw_ref[...], (((1,), (1,)), ((), ())),
            preferred_element_type=jnp.float32)

    xb = xb_ref[...]
    dff = ga_ref.shape[1]
    d_half = dwa_ref.shape[1]

    wg = ga_ref[...].reshape(2 * dff, ga_ref.shape[2])
    wu = ub_ref[...].reshape(2 * dff, ub_ref.shape[2])
    g2 = jax.lax.dot_general(
        xb, wg.astype(jnp.bfloat16), (((1,), (1,)), ((), ())),
        preferred_element_type=jnp.float32)
    u2 = jax.lax.dot_general(
        xb, wu.astype(jnp.bfloat16), (((1,), (1,)), ((), ())),
        preferred_element_type=jnp.float32)

    comb = comb_ref[...]
    lane = jax.lax.broadcasted_iota(jnp.int32, comb.shape, 1)
    s0 = jnp.sum(jnp.where(lane == 2 * i, comb, 0.0), axis=1, keepdims=True)
    s1 = jnp.sum(jnp.where(lane == 2 * i + 1, comb, 0.0), axis=1,
                 keepdims=True)

    g0 = g2[:, :dff]
    g1 = g2[:, dff:]
    u0 = u2[:, :dff]
    u1 = u2[:, dff:]
    h0 = ((g0 * jax.nn.sigmoid(g0) * u0) * s0).astype(jnp.bfloat16)
    h1 = ((g1 * jax.nn.sigmoid(g1) * u1) * s1).astype(jnp.bfloat16)

    ya = jax.lax.dot_general(
        h0, dwa_ref[0].astype(jnp.bfloat16), (((1,), (1,)), ((), ())),
        preferred_element_type=jnp.float32) + jax.lax.dot_general(
        h1, dwa_ref[1].astype(jnp.bfloat16), (((1,), (1,)), ((), ())),
        preferred_element_type=jnp.float32)
    yb = jax.lax.dot_general(
        h0, dwb_ref[0].astype(jnp.bfloat16), (((1,), (1,)), ((), ())),
        preferred_element_type=jnp.float32) + jax.lax.dot_general(
        h1, dwb_ref[1].astype(jnp.bfloat16), (((1,), (1,)), ((), ())),
        preferred_element_type=jnp.float32)
    out_ref[:, :d_half] += ya
    out_ref[:, d_half:] += yb


def kernel(hidden_states, gate_weight, gate_up_weights, down_weights,
           shared_gate_up_weight, shared_down_weight):
    orig_shape = hidden_states.shape
    D = orig_shape[-1]
    x = hidden_states.reshape(-1, D)
    T = x.shape[0]
    E, two_dff, _ = gate_up_weights.shape
    dff = down_weights.shape[2]
    inter = shared_down_weight.shape[1]

    out = pl.pallas_call(
        _moe_kernel,
        grid=(E // 2,),
        in_specs=[
            pl.BlockSpec((T, D), lambda i: (0, 0)),
            pl.BlockSpec((E, D), lambda i: (0, 0)),
            pl.BlockSpec((2, two_dff // 2, D), lambda i: (i, 0, 0)),
            pl.BlockSpec((2, two_dff // 2, D), lambda i: (i, 1, 0)),
            pl.BlockSpec((2, D // 2, dff), lambda i: (i, 0, 0)),
            pl.BlockSpec((2, D // 2, dff), lambda i: (i, 1, 0)),
            pl.BlockSpec((2 * inter, D), lambda i: (0, 0)),
            pl.BlockSpec((D, inter), lambda i: (0, 0)),
        ],
        out_specs=pl.BlockSpec((T, D), lambda i: (0, 0)),
        out_shape=jax.ShapeDtypeStruct((T, D), jnp.float32),
        scratch_shapes=[
            pltpu.VMEM((T, E), jnp.float32),
            pltpu.VMEM((T, D), jnp.bfloat16),
        ],
        compiler_params=pltpu.CompilerParams(
            dimension_semantics=("arbitrary",)),
    )(x, gate_weight, gate_up_weights, gate_up_weights,
      down_weights, down_weights,
      shared_gate_up_weight, shared_down_weight)

    return out.reshape(orig_shape)
